# baseline (speedup 1.0000x reference)
import functools

import jax
import jax.numpy as jnp
from jax import lax
from jax.experimental import pallas as pl
from jax.experimental.pallas import tpu as pltpu


def _enc_body(x_ref, wb_ref, bb_ref, w1_ref, s1_ref):
    xm = jnp.mean(x_ref[...], axis=1, keepdims=True)
    h = xm * wb_ref[...] + bb_ref[...]
    s1 = jnp.dot(h, w1_ref[...], preferred_element_type=jnp.float32)
    s1_ref[...] = s1.astype(jnp.bfloat16)


def _topk_body(alpha_ref, kcap_ref, adj_ref, s1_ref, b1_ref,
               w_ref, h1_ref):
    a = adj_ref[...]
    bn = a.shape
    ones = jnp.ones((bn[1], 128), jnp.bfloat16)

    def count(mask):
        mb = jnp.where(mask, jnp.float32(1),
                       jnp.float32(0)).astype(jnp.bfloat16)
        return jnp.dot(mb, ones,
                       preferred_element_type=jnp.float32)[:, 0:1]

    colid = lax.broadcasted_iota(jnp.int32, bn, 1)
    alpha = alpha_ref[0]
    kk = jnp.minimum(kcap_ref[0], 32).astype(jnp.float32)
    key = lax.bitcast_convert_type(a, jnp.int32)
    p = jnp.zeros((bn[0], 1), jnp.int32)
    for b in range(29, -1, -1):
        t = p + jnp.int32(1 << b)
        c = count(key < t)
        p = jnp.where(c < kk, t, p)
    less = key < p
    eq = key == p
    m = kk - count(less)
    eqcol = jnp.where(eq, colid, jnp.int32(bn[1]))
    q = jnp.zeros((bn[0], 1), jnp.int32)
    for b in range(11, -1, -1):
        t = q + jnp.int32(1 << b)
        c = count(eqcol < t)
        q = jnp.where(c < m, t, q)
    sel = less | (eq & (colid <= q))
    w = jnp.where(sel, jnp.exp(-alpha * a), 0.0)
    z = jnp.sum(w, axis=1, keepdims=True)
    wn = (w / z)
    wb16 = wn.astype(jnp.bfloat16)
    w_ref[...] = wb16
    a1 = jnp.dot(wb16, s1_ref[...],
                 preferred_element_type=jnp.float32) + b1_ref[...]
    h1_ref[...] = jnp.maximum(a1, 0.0).astype(jnp.bfloat16)


def _l2_body(w_ref, h1_ref, w2_ref, b2_ref, out_ref):
    s2 = jnp.dot(h1_ref[...], w2_ref[...].astype(jnp.bfloat16),
                 preferred_element_type=jnp.float32)
    a2 = jnp.dot(w_ref[...], s2.astype(jnp.bfloat16),
                 preferred_element_type=jnp.float32) + b2_ref[...]
    mx = jnp.max(a2, axis=1, keepdims=True)
    e = jnp.exp(a2 - mx)
    lse = jnp.log(jnp.sum(e, axis=1, keepdims=True)) + mx
    out_ref[...] = a2 - lse


def kernel(x, adj, W_b, b_b, W1, b1, W2, b2, K, alpha):
    n, c_in, t = x.shape
    d = W1.shape[0]
    nc = W2.shape[1]
    x2 = x.reshape(n, c_in * t)
    bb = b_b.reshape(1, d)
    b1r = b1.reshape(1, d)
    b2r = b2.reshape(1, nc)
    alpha_f = jnp.asarray(alpha, jnp.float32).reshape(1)
    kcap = jnp.asarray(K, jnp.int32).reshape(1)

    benc = 512
    s1 = pl.pallas_call(
        _enc_body,
        grid=(n // benc,),
        in_specs=[
            pl.BlockSpec((benc, c_in * t), lambda i: (i, 0)),
            pl.BlockSpec((c_in, d), lambda i: (0, 0)),
            pl.BlockSpec((1, d), lambda i: (0, 0)),
            pl.BlockSpec((d, d), lambda i: (0, 0)),
        ],
        out_specs=pl.BlockSpec((benc, d), lambda i: (i, 0)),
        out_shape=jax.ShapeDtypeStruct((n, d), jnp.bfloat16),
    )(x2, W_b, bb, W1)

    btop = 128
    grid_spec = pltpu.PrefetchScalarGridSpec(
        num_scalar_prefetch=2,
        grid=(n // btop,),
        in_specs=[
            pl.BlockSpec((btop, n), lambda i, *_: (i, 0)),
            pl.BlockSpec((n, d), lambda i, *_: (0, 0)),
            pl.BlockSpec((1, d), lambda i, *_: (0, 0)),
        ],
        out_specs=[
            pl.BlockSpec((btop, n), lambda i, *_: (i, 0)),
            pl.BlockSpec((btop, d), lambda i, *_: (i, 0)),
        ],
    )
    wdense, h1 = pl.pallas_call(
        _topk_body,
        grid_spec=grid_spec,
        out_shape=[
            jax.ShapeDtypeStruct((n, n), jnp.bfloat16),
            jax.ShapeDtypeStruct((n, d), jnp.bfloat16),
        ],
    )(alpha_f, kcap, adj, s1, b1r)

    bl2 = 512
    out = pl.pallas_call(
        _l2_body,
        grid=(n // bl2,),
        in_specs=[
            pl.BlockSpec((bl2, n), lambda i: (i, 0)),
            pl.BlockSpec((n, d), lambda i: (0, 0)),
            pl.BlockSpec((d, nc), lambda i: (0, 0)),
            pl.BlockSpec((1, nc), lambda i: (0, 0)),
        ],
        out_specs=pl.BlockSpec((bl2, nc), lambda i: (i, 0)),
        out_shape=jax.ShapeDtypeStruct((n, nc), jnp.float32),
    )(wdense, h1, W2, b2r)
    return out


# speedup vs baseline: 1.8534x; 1.8534x over previous
import functools

import jax
import jax.numpy as jnp
from jax import lax
from jax.experimental import pallas as pl
from jax.experimental.pallas import tpu as pltpu


def _enc_body(x_ref, wb_ref, bb_ref, w1_ref, s1_ref):
    xm = jnp.mean(x_ref[...], axis=1, keepdims=True)
    h = xm * wb_ref[...] + bb_ref[...]
    s1 = jnp.dot(h, w1_ref[...], preferred_element_type=jnp.float32)
    s1_ref[...] = s1.astype(jnp.bfloat16)


def _topk_body(alpha_ref, kcap_ref, adj_ref, s1_ref, b1_ref,
               w_ref, h1_ref):
    a = adj_ref[...]
    bn = a.shape
    def count(mask):
        return jnp.sum(jnp.where(mask, jnp.float32(1), jnp.float32(0)),
                       axis=1, keepdims=True)

    colid = lax.broadcasted_iota(jnp.int32, bn, 1)
    alpha = alpha_ref[0]
    kk = jnp.minimum(kcap_ref[0], 32).astype(jnp.float32)
    key = lax.bitcast_convert_type(a, jnp.int32)
    p = jnp.zeros((bn[0], 1), jnp.int32)
    for b in range(29, -1, -1):
        t = p + jnp.int32(1 << b)
        c = count(key < t)
        p = jnp.where(c < kk, t, p)
    less = key < p
    eq = key == p
    m = kk - count(less)
    eqcol = jnp.where(eq, colid, jnp.int32(bn[1]))
    q = jnp.zeros((bn[0], 1), jnp.int32)
    for b in range(11, -1, -1):
        t = q + jnp.int32(1 << b)
        c = count(eqcol < t)
        q = jnp.where(c < m, t, q)
    sel = less | (eq & (colid <= q))
    w = jnp.where(sel, jnp.exp(-alpha * a), 0.0)
    z = jnp.sum(w, axis=1, keepdims=True)
    wn = (w / z)
    wb16 = wn.astype(jnp.bfloat16)
    w_ref[...] = wb16
    a1 = jnp.dot(wb16, s1_ref[...],
                 preferred_element_type=jnp.float32) + b1_ref[...]
    h1_ref[...] = jnp.maximum(a1, 0.0).astype(jnp.bfloat16)


def _l2_body(w_ref, h1_ref, w2_ref, b2_ref, out_ref):
    s2 = jnp.dot(h1_ref[...], w2_ref[...].astype(jnp.bfloat16),
                 preferred_element_type=jnp.float32)
    a2 = jnp.dot(w_ref[...], s2.astype(jnp.bfloat16),
                 preferred_element_type=jnp.float32) + b2_ref[...]
    mx = jnp.max(a2, axis=1, keepdims=True)
    e = jnp.exp(a2 - mx)
    lse = jnp.log(jnp.sum(e, axis=1, keepdims=True)) + mx
    out_ref[...] = a2 - lse


def kernel(x, adj, W_b, b_b, W1, b1, W2, b2, K, alpha):
    n, c_in, t = x.shape
    d = W1.shape[0]
    nc = W2.shape[1]
    x2 = x.reshape(n, c_in * t)
    bb = b_b.reshape(1, d)
    b1r = b1.reshape(1, d)
    b2r = b2.reshape(1, nc)
    alpha_f = jnp.asarray(alpha, jnp.float32).reshape(1)
    kcap = jnp.asarray(K, jnp.int32).reshape(1)

    benc = 512
    s1 = pl.pallas_call(
        _enc_body,
        grid=(n // benc,),
        in_specs=[
            pl.BlockSpec((benc, c_in * t), lambda i: (i, 0)),
            pl.BlockSpec((c_in, d), lambda i: (0, 0)),
            pl.BlockSpec((1, d), lambda i: (0, 0)),
            pl.BlockSpec((d, d), lambda i: (0, 0)),
        ],
        out_specs=pl.BlockSpec((benc, d), lambda i: (i, 0)),
        out_shape=jax.ShapeDtypeStruct((n, d), jnp.bfloat16),
    )(x2, W_b, bb, W1)

    btop = 128
    grid_spec = pltpu.PrefetchScalarGridSpec(
        num_scalar_prefetch=2,
        grid=(n // btop,),
        in_specs=[
            pl.BlockSpec((btop, n), lambda i, *_: (i, 0)),
            pl.BlockSpec((n, d), lambda i, *_: (0, 0)),
            pl.BlockSpec((1, d), lambda i, *_: (0, 0)),
        ],
        out_specs=[
            pl.BlockSpec((btop, n), lambda i, *_: (i, 0)),
            pl.BlockSpec((btop, d), lambda i, *_: (i, 0)),
        ],
    )
    wdense, h1 = pl.pallas_call(
        _topk_body,
        grid_spec=grid_spec,
        out_shape=[
            jax.ShapeDtypeStruct((n, n), jnp.bfloat16),
            jax.ShapeDtypeStruct((n, d), jnp.bfloat16),
        ],
    )(alpha_f, kcap, adj, s1, b1r)

    bl2 = 512
    out = pl.pallas_call(
        _l2_body,
        grid=(n // bl2,),
        in_specs=[
            pl.BlockSpec((bl2, n), lambda i: (i, 0)),
            pl.BlockSpec((n, d), lambda i: (0, 0)),
            pl.BlockSpec((d, nc), lambda i: (0, 0)),
            pl.BlockSpec((1, nc), lambda i: (0, 0)),
        ],
        out_specs=pl.BlockSpec((bl2, nc), lambda i: (i, 0)),
        out_shape=jax.ShapeDtypeStruct((n, nc), jnp.float32),
    )(wdense, h1, W2, b2r)
    return out


# speedup vs baseline: 2.2158x; 1.1955x over previous
import functools

import jax
import jax.numpy as jnp
from jax import lax
from jax.experimental import pallas as pl
from jax.experimental.pallas import tpu as pltpu


def _enc_body(x_ref, wb_ref, bb_ref, w1_ref, s1_ref):
    xm = jnp.mean(x_ref[...], axis=1, keepdims=True)
    h = xm * wb_ref[...] + bb_ref[...]
    s1 = jnp.dot(h, w1_ref[...], preferred_element_type=jnp.float32)
    s1_ref[...] = s1.astype(jnp.bfloat16)


def _topk_body(alpha_ref, kcap_ref, adj_ref, s1_ref, b1_ref,
               w_ref, h1_ref, q_ref):
    a = adj_ref[...]
    bn = a.shape
    def count(mask):
        return jnp.sum(jnp.where(mask, jnp.float32(1), jnp.float32(0)),
                       axis=1, keepdims=True)

    colid = lax.broadcasted_iota(jnp.int32, bn, 1)
    alpha = alpha_ref[0]
    kk = jnp.minimum(kcap_ref[0], 32).astype(jnp.float32)
    key = lax.bitcast_convert_type(a, jnp.int32)
    p = jnp.zeros((bn[0], 1), jnp.int32)
    for b in range(29, -1, -1):
        t = p + jnp.int32(1 << b)
        c = count(key < t)
        p = jnp.where(c < kk, t, p)
    less = key < p
    eq = key == p
    m = kk - count(less)
    q_ref[...] = jnp.full((bn[0], 1), jnp.int32(bn[1]))
    excess = jnp.max(count(eq) - m) > 0.0

    @pl.when(excess)
    def _tie_break():
        eqcol = jnp.where(eq, colid, jnp.int32(bn[1]))
        q = jnp.zeros((bn[0], 1), jnp.int32)
        for b in range(11, -1, -1):
            t = q + jnp.int32(1 << b)
            c = count(eqcol < t)
            q = jnp.where(c < m, t, q)
        q_ref[...] = q

    sel = less | (eq & (colid <= q_ref[...]))
    w = jnp.where(sel, jnp.exp(-alpha * a), 0.0)
    z = jnp.sum(w, axis=1, keepdims=True)
    wn = (w / z)
    wb16 = wn.astype(jnp.bfloat16)
    w_ref[...] = wb16
    a1 = jnp.dot(wb16, s1_ref[...],
                 preferred_element_type=jnp.float32) + b1_ref[...]
    h1_ref[...] = jnp.maximum(a1, 0.0).astype(jnp.bfloat16)


def _l2_body(w_ref, h1_ref, w2_ref, b2_ref, out_ref):
    s2 = jnp.dot(h1_ref[...], w2_ref[...].astype(jnp.bfloat16),
                 preferred_element_type=jnp.float32)
    a2 = jnp.dot(w_ref[...], s2.astype(jnp.bfloat16),
                 preferred_element_type=jnp.float32) + b2_ref[...]
    mx = jnp.max(a2, axis=1, keepdims=True)
    e = jnp.exp(a2 - mx)
    lse = jnp.log(jnp.sum(e, axis=1, keepdims=True)) + mx
    out_ref[...] = a2 - lse


def kernel(x, adj, W_b, b_b, W1, b1, W2, b2, K, alpha):
    n, c_in, t = x.shape
    d = W1.shape[0]
    nc = W2.shape[1]
    x2 = x.reshape(n, c_in * t)
    bb = b_b.reshape(1, d)
    b1r = b1.reshape(1, d)
    b2r = b2.reshape(1, nc)
    alpha_f = jnp.asarray(alpha, jnp.float32).reshape(1)
    kcap = jnp.asarray(K, jnp.int32).reshape(1)

    benc = 512
    s1 = pl.pallas_call(
        _enc_body,
        grid=(n // benc,),
        in_specs=[
            pl.BlockSpec((benc, c_in * t), lambda i: (i, 0)),
            pl.BlockSpec((c_in, d), lambda i: (0, 0)),
            pl.BlockSpec((1, d), lambda i: (0, 0)),
            pl.BlockSpec((d, d), lambda i: (0, 0)),
        ],
        out_specs=pl.BlockSpec((benc, d), lambda i: (i, 0)),
        out_shape=jax.ShapeDtypeStruct((n, d), jnp.bfloat16),
    )(x2, W_b, bb, W1)

    btop = 128
    grid_spec = pltpu.PrefetchScalarGridSpec(
        num_scalar_prefetch=2,
        grid=(n // btop,),
        in_specs=[
            pl.BlockSpec((btop, n), lambda i, *_: (i, 0)),
            pl.BlockSpec((n, d), lambda i, *_: (0, 0)),
            pl.BlockSpec((1, d), lambda i, *_: (0, 0)),
        ],
        out_specs=[
            pl.BlockSpec((btop, n), lambda i, *_: (i, 0)),
            pl.BlockSpec((btop, d), lambda i, *_: (i, 0)),
        ],
        scratch_shapes=[pltpu.VMEM((btop, 1), jnp.int32)],
    )
    wdense, h1 = pl.pallas_call(
        _topk_body,
        grid_spec=grid_spec,
        out_shape=[
            jax.ShapeDtypeStruct((n, n), jnp.bfloat16),
            jax.ShapeDtypeStruct((n, d), jnp.bfloat16),
        ],
    )(alpha_f, kcap, adj, s1, b1r)

    bl2 = 512
    out = pl.pallas_call(
        _l2_body,
        grid=(n // bl2,),
        in_specs=[
            pl.BlockSpec((bl2, n), lambda i: (i, 0)),
            pl.BlockSpec((n, d), lambda i: (0, 0)),
            pl.BlockSpec((d, nc), lambda i: (0, 0)),
            pl.BlockSpec((1, nc), lambda i: (0, 0)),
        ],
        out_specs=pl.BlockSpec((bl2, nc), lambda i: (i, 0)),
        out_shape=jax.ShapeDtypeStruct((n, nc), jnp.float32),
    )(wdense, h1, W2, b2r)
    return out


# speedup vs baseline: 2.5902x; 1.1690x over previous
import functools

import jax
import jax.numpy as jnp
from jax import lax
from jax.experimental import pallas as pl
from jax.experimental.pallas import tpu as pltpu


def _enc_body(x_ref, wb_ref, bb_ref, w1_ref, s1_ref):
    xm = jnp.mean(x_ref[...], axis=1, keepdims=True)
    h = xm * wb_ref[...] + bb_ref[...]
    s1 = jnp.dot(h, w1_ref[...], preferred_element_type=jnp.float32)
    s1_ref[...] = s1.astype(jnp.bfloat16)


def _topk_body(alpha_ref, kcap_ref, adj_ref, s1_ref, b1_ref,
               w_ref, h1_ref, q_ref):
    a = adj_ref[...]
    bn = a.shape
    def count(mask):
        return jnp.sum(jnp.where(mask, jnp.float32(1), jnp.float32(0)),
                       axis=1, keepdims=True)

    def count16(mask16):
        v = jnp.where(mask16, jnp.int16(1), jnp.int16(0))
        width = bn[1]
        while width > 128:
            half = width // 2
            v = v[:, :half] + v[:, half:]
            width = half
        return jnp.sum(v.astype(jnp.int32), axis=1, keepdims=True)

    colid = lax.broadcasted_iota(jnp.int32, bn, 1)
    alpha = alpha_ref[0]
    kk32 = jnp.minimum(kcap_ref[0], 32)
    kk = kk32.astype(jnp.float32)
    key = lax.bitcast_convert_type(a, jnp.int32)
    hi16 = (key >> 15).astype(jnp.int16)
    p = jnp.zeros((bn[0], 1), jnp.int32)
    for b in range(14, -1, -1):
        t = p + jnp.int32(1 << b)
        c = count16(hi16 < t.astype(jnp.int16))
        p = jnp.where(c < kk32, t, p)
    p16 = p.astype(jnp.int16)
    match = hi16 == p16
    m2 = kk32 - count16(hi16 < p16)
    low16 = jnp.where(match, (key & jnp.int32(0x7FFF)).astype(jnp.int16),
                      jnp.int16(0x7FFF))
    pl_ = jnp.zeros((bn[0], 1), jnp.int32)
    for b in range(14, -1, -1):
        t = pl_ + jnp.int32(1 << b)
        c = count16(low16 < t.astype(jnp.int16))
        pl_ = jnp.where(c < m2, t, pl_)
    thresh = (p << 15) | pl_
    less = key < thresh
    eq = key == thresh
    m = kk - count(less)
    q_ref[...] = jnp.full((bn[0], 1), jnp.int32(bn[1]))
    excess = jnp.max(count(eq) - m) > 0.0

    @pl.when(excess)
    def _tie_break():
        eqcol = jnp.where(eq, colid, jnp.int32(bn[1]))
        q = jnp.zeros((bn[0], 1), jnp.int32)
        for b in range(11, -1, -1):
            t = q + jnp.int32(1 << b)
            c = count(eqcol < t)
            q = jnp.where(c < m, t, q)
        q_ref[...] = q

    sel = less | (eq & (colid <= q_ref[...]))
    w = jnp.where(sel, jnp.exp(-alpha * a), 0.0)
    z = jnp.sum(w, axis=1, keepdims=True)
    wn = (w / z)
    wb16 = wn.astype(jnp.bfloat16)
    w_ref[...] = wb16
    a1 = jnp.dot(wb16, s1_ref[...],
                 preferred_element_type=jnp.float32) + b1_ref[...]
    h1_ref[...] = jnp.maximum(a1, 0.0).astype(jnp.bfloat16)


def _l2_body(w_ref, h1_ref, w2_ref, b2_ref, out_ref):
    s2 = jnp.dot(h1_ref[...], w2_ref[...].astype(jnp.bfloat16),
                 preferred_element_type=jnp.float32)
    a2 = jnp.dot(w_ref[...], s2.astype(jnp.bfloat16),
                 preferred_element_type=jnp.float32) + b2_ref[...]
    mx = jnp.max(a2, axis=1, keepdims=True)
    e = jnp.exp(a2 - mx)
    lse = jnp.log(jnp.sum(e, axis=1, keepdims=True)) + mx
    out_ref[...] = a2 - lse


def kernel(x, adj, W_b, b_b, W1, b1, W2, b2, K, alpha):
    n, c_in, t = x.shape
    d = W1.shape[0]
    nc = W2.shape[1]
    x2 = x.reshape(n, c_in * t)
    bb = b_b.reshape(1, d)
    b1r = b1.reshape(1, d)
    b2r = b2.reshape(1, nc)
    alpha_f = jnp.asarray(alpha, jnp.float32).reshape(1)
    kcap = jnp.asarray(K, jnp.int32).reshape(1)

    benc = 512
    s1 = pl.pallas_call(
        _enc_body,
        grid=(n // benc,),
        in_specs=[
            pl.BlockSpec((benc, c_in * t), lambda i: (i, 0)),
            pl.BlockSpec((c_in, d), lambda i: (0, 0)),
            pl.BlockSpec((1, d), lambda i: (0, 0)),
            pl.BlockSpec((d, d), lambda i: (0, 0)),
        ],
        out_specs=pl.BlockSpec((benc, d), lambda i: (i, 0)),
        out_shape=jax.ShapeDtypeStruct((n, d), jnp.bfloat16),
    )(x2, W_b, bb, W1)

    btop = 128
    grid_spec = pltpu.PrefetchScalarGridSpec(
        num_scalar_prefetch=2,
        grid=(n // btop,),
        in_specs=[
            pl.BlockSpec((btop, n), lambda i, *_: (i, 0)),
            pl.BlockSpec((n, d), lambda i, *_: (0, 0)),
            pl.BlockSpec((1, d), lambda i, *_: (0, 0)),
        ],
        out_specs=[
            pl.BlockSpec((btop, n), lambda i, *_: (i, 0)),
            pl.BlockSpec((btop, d), lambda i, *_: (i, 0)),
        ],
        scratch_shapes=[pltpu.VMEM((btop, 1), jnp.int32)],
    )
    wdense, h1 = pl.pallas_call(
        _topk_body,
        grid_spec=grid_spec,
        out_shape=[
            jax.ShapeDtypeStruct((n, n), jnp.bfloat16),
            jax.ShapeDtypeStruct((n, d), jnp.bfloat16),
        ],
    )(alpha_f, kcap, adj, s1, b1r)

    bl2 = 512
    out = pl.pallas_call(
        _l2_body,
        grid=(n // bl2,),
        in_specs=[
            pl.BlockSpec((bl2, n), lambda i: (i, 0)),
            pl.BlockSpec((n, d), lambda i: (0, 0)),
            pl.BlockSpec((d, nc), lambda i: (0, 0)),
            pl.BlockSpec((1, nc), lambda i: (0, 0)),
        ],
        out_specs=pl.BlockSpec((bl2, nc), lambda i: (i, 0)),
        out_shape=jax.ShapeDtypeStruct((n, nc), jnp.float32),
    )(wdense, h1, W2, b2r)
    return out
